# baseline (speedup 1.0000x reference)
import functools

import jax
import jax.numpy as jnp
from jax import lax
from jax.experimental import pallas as pl
from jax.experimental.pallas import tpu as pltpu
from jax.experimental.pallas import tpu_sc as plsc

DIM = 128
K = 16
TAB_D = 128

_NC = 2
_NS = 16


def _sc_gather(tab, idx, chunk):
    E = idx.shape[0]
    D = tab.shape[1]
    nw = _NC * _NS
    epw = E // nw
    nch = epw // chunk

    mesh = plsc.VectorSubcoreMesh(core_axis_name="c", subcore_axis_name="s")

    @functools.partial(
        pl.kernel,
        out_type=jax.ShapeDtypeStruct((E, D), jnp.float32),
        mesh=mesh,
        scratch_types=[
            pltpu.VMEM((epw,), jnp.int32),
            pltpu.VMEM((chunk, D), jnp.float32),
            pltpu.VMEM((chunk, D), jnp.float32),
            pltpu.SemaphoreType.DMA,
            pltpu.SemaphoreType.DMA,
            pltpu.SemaphoreType.DMA,
            pltpu.SemaphoreType.DMA,
        ],
    )
    def gk(tab_hbm, idx_hbm, out_hbm, idx_v, buf0, buf1, rs0, rs1, ws0, ws1):
        wid = lax.axis_index("s") * _NC + lax.axis_index("c")
        base = wid * epw
        pltpu.sync_copy(idx_hbm.at[pl.ds(base, epw)], idx_v)
        bufs = (buf0, buf1)
        rsems = (rs0, rs1)
        wsems = (ws0, ws1)
        rdesc = [None, None]
        wdesc = [None, None]
        rdesc[0] = pltpu.async_copy(tab_hbm.at[idx_v.at[pl.ds(0, chunk)]],
                                    buf0, rs0)
        for g in range(nch):
            b = g % 2
            nb = (g + 1) % 2
            if g + 1 < nch:
                if wdesc[nb] is not None:
                    wdesc[nb].wait()
                rdesc[nb] = pltpu.async_copy(
                    tab_hbm.at[idx_v.at[pl.ds((g + 1) * chunk, chunk)]],
                    bufs[nb], rsems[nb])
            rdesc[b].wait()
            wdesc[b] = pltpu.async_copy(
                bufs[b], out_hbm.at[pl.ds(base + g * chunk, chunk)], wsems[b])
        wdesc[(nch - 1) % 2].wait()
        if nch > 1:
            wdesc[(nch - 2) % 2].wait()

    return gk(tab, idx)


def _silu(x):
    return x * (1.0 / (1.0 + jnp.exp(-x)))


def _unpack_feat(words):
    wi = lax.bitcast_convert_type(words, jnp.int32)
    lo = lax.bitcast_convert_type(lax.shift_left(wi, 16), jnp.float32)
    hi = lax.bitcast_convert_type(
        jnp.bitwise_and(wi, jnp.int32(-65536)), jnp.float32)
    return jnp.concatenate([lo, hi], axis=1).astype(jnp.bfloat16)


def _tc_body(xp_ref, xg_ref, co_ref, w27_ref, posw2T_ref,
             posb2_ref, wqT_ref, wkT_ref, wvT_ref, w1T_ref, b1_ref, w2T_ref,
             woT_ref, bo_ref, out_ref):
    bn = xp_ref.shape[0]
    e = xg_ref.shape[0]
    f32 = jnp.float32
    bf16 = jnp.bfloat16

    xc = _unpack_feat(xp_ref[:, :64])
    cpos = xp_ref[:, 64:67]
    nf = _unpack_feat(xg_ref[:, :64])
    npos = xg_ref[:, 64:67]

    cpos_e = jnp.reshape(jnp.broadcast_to(cpos[:, None, :], (bn, K, 3)), (e, 3))
    rel = npos - cpos_e

    eye3 = jnp.eye(3, dtype=f32)
    relT = lax.dot_general(eye3, rel, (((1,), (1,)), ((), ())),
                           preferred_element_type=f32)
    xs = jnp.reshape(jnp.broadcast_to(relT[:, None, :], (3, 8, e)), (24, e))
    s = (xs - co_ref[:, 0:1]) * co_ref[:, 1:2]
    a = jnp.abs(s)
    near = 2.0 / 3.0 - s * s + a * a * a * 0.5
    t2 = jnp.maximum(2.0 - a, 0.0)
    far = t2 * t2 * t2 * (1.0 / 6.0)
    b24 = jnp.where(a < 1.0, near, far)
    featsT = jnp.concatenate([_silu(relT), b24], axis=0).astype(bf16)
    h = lax.dot_general(featsT, w27_ref[...], (((0,), (0,)), ((), ())),
                        preferred_element_type=f32)
    pe = jnp.dot(jnp.maximum(h, 0.0).astype(bf16), posw2T_ref[...],
                 preferred_element_type=f32) + posb2_ref[...]

    q = jnp.dot(xc, wqT_ref[...], preferred_element_type=f32)
    q_e = jnp.reshape(jnp.broadcast_to(q[:, None, :], (bn, K, DIM)), (e, DIM))
    kf = jnp.dot(nf, wkT_ref[...], preferred_element_type=f32)
    v = jnp.dot(nf, wvT_ref[...], preferred_element_type=f32) + pe

    energy = (q_e - kf + pe).astype(bf16)
    t = jnp.maximum(jnp.dot(energy, w1T_ref[...], preferred_element_type=f32)
                    + b1_ref[...], 0.0)
    a = jnp.dot(t.astype(bf16), w2T_ref[...], preferred_element_type=f32)

    a3 = jnp.reshape(a, (bn, K, DIM))
    m = jnp.max(a3, axis=1, keepdims=True)
    ex = jnp.exp(a3 - m)
    s = jnp.sum(ex, axis=1, keepdims=True)
    p = ex / s
    v3 = jnp.reshape(v, (bn, K, DIM))
    o = jnp.sum(p * v3, axis=1)

    out_ref[...] = jnp.maximum(
        jnp.dot(o.astype(bf16), woT_ref[...], preferred_element_type=f32)
        + bo_ref[...], 0.0)


def _tc_dense(xp, xg, co, w27, posw2T, posb2, wqT, wkT, wvT, w1T,
              b1, w2T, woT, bo, bn):
    n = xp.shape[0]
    full = lambda shape: pl.BlockSpec(shape, lambda i: (0, 0))
    return pl.pallas_call(
        _tc_body,
        grid=(n // bn,),
        in_specs=[
            pl.BlockSpec((bn, TAB_D), lambda i: (i, 0)),
            pl.BlockSpec((bn * K, TAB_D), lambda i: (i, 0)),
            full(co.shape),
            full(w27.shape),
            full(posw2T.shape),
            full(posb2.shape),
            full(wqT.shape),
            full(wkT.shape),
            full(wvT.shape),
            full(w1T.shape),
            full(b1.shape),
            full(w2T.shape),
            full(woT.shape),
            full(bo.shape),
        ],
        out_specs=pl.BlockSpec((bn, DIM), lambda i: (i, 0)),
        out_shape=jax.ShapeDtypeStruct((n, DIM), jnp.float32),
        compiler_params=pltpu.CompilerParams(
            dimension_semantics=("parallel",)),
    )(xp, xg, co, w27, posw2T, posb2, wqT, wkT, wvT, w1T, b1, w2T, woT, bo)


def kernel(x, pos, knn_idx, kan_base_w, kan_spline_w, kan_grid, pos_w2,
           pos_b2, attn_w1, attn_b1, attn_w2, attn_b2, wq, wk, wv, wo, bo):
    del attn_b2
    bsz, n, c = x.shape
    bf16 = jnp.bfloat16
    x2 = x[0]
    pos2 = pos[0].astype(jnp.float32)
    xb = x2.astype(bf16)
    lo16 = lax.bitcast_convert_type(xb[:, :64], jnp.uint16).astype(jnp.uint32)
    hi16 = lax.bitcast_convert_type(xb[:, 64:], jnp.uint16).astype(jnp.uint32)
    xw = lax.bitcast_convert_type(
        jnp.bitwise_or(jnp.left_shift(hi16, 16), lo16), jnp.float32)
    pad = jnp.zeros((n, TAB_D - 64 - 3), jnp.float32)
    tab = jnp.concatenate([xw, pos2, pad], axis=1)
    idx = knn_idx[0].reshape(-1).astype(jnp.int32)

    xg = _sc_gather(tab, idx, chunk=200)

    w27 = jnp.concatenate(
        [kan_base_w, jnp.reshape(kan_spline_w, (DIM, 24))], axis=1).T

    step = kan_grid[:, 1:2] - kan_grid[:, 0:1]
    centers = kan_grid[:, :8] + 2.0 * step
    invh = jnp.broadcast_to(1.0 / step, (3, 8))
    co = jnp.stack([centers.reshape(24), invh.reshape(24)], axis=1)

    out = _tc_dense(
        tab, xg,
        co,
        w27.astype(bf16),
        pos_w2.T.astype(bf16),
        pos_b2[None, :],
        wq.T.astype(bf16), wk.T.astype(bf16), wv.T.astype(bf16),
        attn_w1.T.astype(bf16),
        attn_b1[None, :],
        attn_w2.T.astype(bf16),
        wo.T.astype(bf16),
        bo[None, :],
        bn=200,
    )
    return out[None]


# speedup vs baseline: 1.0584x; 1.0584x over previous
import functools

import jax
import jax.numpy as jnp
from jax import lax
from jax.experimental import pallas as pl
from jax.experimental.pallas import tpu as pltpu
from jax.experimental.pallas import tpu_sc as plsc

DIM = 128
K = 16
TAB_D = 128

_NC = 2
_NS = 16


def _sc_gather(tab, idx, chunk):
    E = idx.shape[0]
    D = tab.shape[1]
    nw = _NC * _NS
    epw = E // nw
    nch = epw // chunk

    mesh = plsc.VectorSubcoreMesh(core_axis_name="c", subcore_axis_name="s")

    @functools.partial(
        pl.kernel,
        out_type=jax.ShapeDtypeStruct((E, D), jnp.float32),
        mesh=mesh,
        scratch_types=[
            pltpu.VMEM((epw,), jnp.int32),
            pltpu.VMEM((chunk, D), jnp.float32),
            pltpu.VMEM((chunk, D), jnp.float32),
            pltpu.SemaphoreType.DMA,
            pltpu.SemaphoreType.DMA,
            pltpu.SemaphoreType.DMA,
            pltpu.SemaphoreType.DMA,
        ],
    )
    def gk(tab_hbm, idx_hbm, out_hbm, idx_v, buf0, buf1, rs0, rs1, ws0, ws1):
        wid = lax.axis_index("s") * _NC + lax.axis_index("c")
        base = wid * epw
        pltpu.sync_copy(idx_hbm.at[pl.ds(base, epw)], idx_v)
        bufs = (buf0, buf1)
        rsems = (rs0, rs1)
        wsems = (ws0, ws1)
        rdesc = [None, None]
        wdesc = [None, None]
        rdesc[0] = pltpu.async_copy(tab_hbm.at[idx_v.at[pl.ds(0, chunk)]],
                                    buf0, rs0)
        for g in range(nch):
            b = g % 2
            nb = (g + 1) % 2
            if g + 1 < nch:
                if wdesc[nb] is not None:
                    wdesc[nb].wait()
                rdesc[nb] = pltpu.async_copy(
                    tab_hbm.at[idx_v.at[pl.ds((g + 1) * chunk, chunk)]],
                    bufs[nb], rsems[nb])
            rdesc[b].wait()
            wdesc[b] = pltpu.async_copy(
                bufs[b], out_hbm.at[pl.ds(base + g * chunk, chunk)], wsems[b])
        wdesc[(nch - 1) % 2].wait()
        if nch > 1:
            wdesc[(nch - 2) % 2].wait()

    return gk(tab, idx)


def _silu(x):
    return x * (1.0 / (1.0 + jnp.exp(-x)))


def _unpack_feat(words):
    wi = lax.bitcast_convert_type(words, jnp.int32)
    lo = lax.bitcast_convert_type(lax.shift_left(wi, 16), jnp.float32)
    hi = lax.bitcast_convert_type(
        jnp.bitwise_and(wi, jnp.int32(-65536)), jnp.float32)
    return jnp.concatenate([lo, hi], axis=1).astype(jnp.bfloat16)


def _tc_body(xp_ref, xg_ref, co_ref, w27_ref, posw2T_ref,
             posb2_ref, wqT_ref, wkT_ref, wvT_ref, w1T_ref, b1_ref, w2T_ref,
             woT_ref, bo_ref, out_ref):
    bn = xp_ref.shape[0]
    e = xg_ref.shape[0]
    f32 = jnp.float32
    bf16 = jnp.bfloat16

    xc = _unpack_feat(xp_ref[:, :64])
    cpos = xp_ref[:, 64:67]
    nf = _unpack_feat(xg_ref[:, :64])
    npos = xg_ref[:, 64:67]

    cpos_e = jnp.reshape(jnp.broadcast_to(cpos[:, None, :], (bn, K, 3)), (e, 3))
    rel = npos - cpos_e

    eye3 = jnp.eye(3, dtype=f32)
    relT = lax.dot_general(eye3, rel, (((1,), (1,)), ((), ())),
                           preferred_element_type=f32)
    xs = jnp.reshape(jnp.broadcast_to(relT[:, None, :], (3, 8, e)), (24, e))
    s = (xs - co_ref[:, 0:1]) * co_ref[:, 1:2]
    a = jnp.abs(s)
    near = 2.0 / 3.0 - s * s + a * a * a * 0.5
    t2 = jnp.maximum(2.0 - a, 0.0)
    far = t2 * t2 * t2 * (1.0 / 6.0)
    b24 = jnp.where(a < 1.0, near, far)
    featsT = jnp.concatenate([_silu(relT), b24], axis=0).astype(bf16)
    h = lax.dot_general(featsT, w27_ref[...], (((0,), (0,)), ((), ())),
                        preferred_element_type=f32)
    pe = jnp.dot(jnp.maximum(h, 0.0).astype(bf16), posw2T_ref[...],
                 preferred_element_type=f32) + posb2_ref[...]

    q = jnp.dot(xc, wqT_ref[...], preferred_element_type=f32)
    q_e = jnp.reshape(jnp.broadcast_to(q[:, None, :], (bn, K, DIM)), (e, DIM))
    kf = jnp.dot(nf, wkT_ref[...], preferred_element_type=f32)
    v = jnp.dot(nf, wvT_ref[...], preferred_element_type=f32) + pe

    energy = (q_e - kf + pe).astype(bf16)
    t = jnp.maximum(jnp.dot(energy, w1T_ref[...], preferred_element_type=f32)
                    + b1_ref[...], 0.0)
    a = jnp.dot(t.astype(bf16), w2T_ref[...], preferred_element_type=f32)

    a3 = jnp.reshape(a, (bn, K, DIM))
    m = jnp.max(a3, axis=1, keepdims=True)
    ex = jnp.exp(a3 - m)
    s = jnp.sum(ex, axis=1, keepdims=True)
    p = ex / s
    v3 = jnp.reshape(v, (bn, K, DIM))
    o = jnp.sum(p * v3, axis=1)

    out_ref[...] = jnp.maximum(
        jnp.dot(o.astype(bf16), woT_ref[...], preferred_element_type=f32)
        + bo_ref[...], 0.0)


def _tc_dense(xp, xg, co, w27, posw2T, posb2, wqT, wkT, wvT, w1T,
              b1, w2T, woT, bo, bn):
    n = xp.shape[0]
    full = lambda shape: pl.BlockSpec(shape, lambda i: (0, 0))
    return pl.pallas_call(
        _tc_body,
        grid=(n // bn,),
        in_specs=[
            pl.BlockSpec((bn, TAB_D), lambda i: (i, 0)),
            pl.BlockSpec((bn * K, TAB_D), lambda i: (i, 0)),
            full(co.shape),
            full(w27.shape),
            full(posw2T.shape),
            full(posb2.shape),
            full(wqT.shape),
            full(wkT.shape),
            full(wvT.shape),
            full(w1T.shape),
            full(b1.shape),
            full(w2T.shape),
            full(woT.shape),
            full(bo.shape),
        ],
        out_specs=pl.BlockSpec((bn, DIM), lambda i: (i, 0)),
        out_shape=jax.ShapeDtypeStruct((n, DIM), jnp.float32),
        compiler_params=pltpu.CompilerParams(
            dimension_semantics=("parallel",)),
    )(xp, xg, co, w27, posw2T, posb2, wqT, wkT, wvT, w1T, b1, w2T, woT, bo)


def kernel(x, pos, knn_idx, kan_base_w, kan_spline_w, kan_grid, pos_w2,
           pos_b2, attn_w1, attn_b1, attn_w2, attn_b2, wq, wk, wv, wo, bo):
    del attn_b2
    bsz, n, c = x.shape
    bf16 = jnp.bfloat16
    x2 = x[0]
    pos2 = pos[0].astype(jnp.float32)
    xb = x2.astype(bf16)
    lo16 = lax.bitcast_convert_type(xb[:, :64], jnp.uint16).astype(jnp.uint32)
    hi16 = lax.bitcast_convert_type(xb[:, 64:], jnp.uint16).astype(jnp.uint32)
    xw = lax.bitcast_convert_type(
        jnp.bitwise_or(jnp.left_shift(hi16, 16), lo16), jnp.float32)
    pad = jnp.zeros((n, TAB_D - 64 - 3), jnp.float32)
    tab = jnp.concatenate([xw, pos2, pad], axis=1)
    idx = knn_idx[0].reshape(-1).astype(jnp.int32)

    xg = _sc_gather(tab, idx, chunk=200)

    w27 = jnp.concatenate(
        [kan_base_w, jnp.reshape(kan_spline_w, (DIM, 24))], axis=1).T

    step = kan_grid[:, 1:2] - kan_grid[:, 0:1]
    centers = kan_grid[:, :8] + 2.0 * step
    invh = jnp.broadcast_to(1.0 / step, (3, 8))
    co = jnp.stack([centers.reshape(24), invh.reshape(24)], axis=1)

    out = _tc_dense(
        tab, xg,
        co,
        w27.astype(bf16),
        pos_w2.T.astype(bf16),
        pos_b2[None, :],
        wq.T.astype(bf16), wk.T.astype(bf16), wv.T.astype(bf16),
        attn_w1.T.astype(bf16),
        attn_b1[None, :],
        attn_w2.T.astype(bf16),
        wo.T.astype(bf16),
        bo[None, :],
        bn=1000,
    )
    return out[None]
